# baseline (speedup 1.0000x reference)
import jax
import jax.numpy as jnp
from jax import lax
from jax.experimental import pallas as pl
from jax.experimental.pallas import tpu as pltpu
from jax.experimental.pallas import tpu_sc as plsc

NC = 2
NS = 16
L = 16
NW = NC * NS

BATCH = 16384
D = 32
ROWS_PER_SUP = 128 // D
B_PER_W = BATCH // NW
CHUNK = 128
N_CHUNKS = B_PER_W // CHUNK
G_PER_CHUNK = CHUNK // L
N_GROUPS = B_PER_W // L


def _gmf_body(u_tab, i_tab, u_idx, i_idx, wb,
              out,
              u_idx_v, i_idx_v, u_sup_idx, i_sup_idx, u_off_v, i_off_v,
              u_sup, i_sup, wb_v, prod_t, out_v, sem):
    wid = lax.axis_index("s") * NC + lax.axis_index("c")
    cbase = wid * N_CHUNKS

    pltpu.sync_copy(u_idx.at[pl.ds(cbase, N_CHUNKS)], u_idx_v)
    pltpu.sync_copy(i_idx.at[pl.ds(cbase, N_CHUNKS)], i_idx_v)
    pltpu.sync_copy(wb, wb_v)

    for k in range(N_CHUNKS):
        for j in range(G_PER_CHUNK):
            s = pl.ds(j * L, L)
            uv = u_idx_v[k, s]
            iv = i_idx_v[k, s]
            u_sup_idx[k, s] = uv >> 2
            i_sup_idx[k, s] = iv >> 2
            u_off_v[pl.ds(k * CHUNK + j * L, L)] = (uv & 3) << 5
            i_off_v[pl.ds(k * CHUNK + j * L, L)] = (iv & 3) << 5

    w_lo = wb_v[0, pl.ds(0, L)]
    w_hi = wb_v[1, pl.ds(0, L)]
    b_vec = wb_v[2, pl.ds(0, L)]
    lane = jnp.arange(L, dtype=jnp.int32)
    sc_lo = lane * B_PER_W
    sc_hi = sc_lo + L * B_PER_W

    for c in range(N_CHUNKS):
        cp_u = pltpu.async_copy(u_tab.at[u_sup_idx.at[c]], u_sup, sem)
        cp_i = pltpu.async_copy(i_tab.at[i_sup_idx.at[c]], i_sup, sem)
        cp_u.wait()
        cp_i.wait()

        def row_group(g, carry, c=c):
            uoff = u_off_v[pl.ds(c * CHUNK + g * L, L)]
            ioff = i_off_v[pl.ds(c * CHUNK + g * L, L)]
            for j in range(L):
                r_loc = g * L + j
                r_glob = c * CHUNK + r_loc
                uo = uoff[j]
                io = ioff[j]
                u0 = u_sup[r_loc, pl.ds(uo, L)]
                u1 = u_sup[r_loc, pl.ds(uo + L, L)]
                i0 = i_sup[r_loc, pl.ds(io, L)]
                i1 = i_sup[r_loc, pl.ds(io + L, L)]
                p0 = u0 * i0 * w_lo
                p1 = u1 * i1 * w_hi
                plsc.store_scatter(prod_t, [sc_lo + r_glob], p0)
                plsc.store_scatter(prod_t, [sc_hi + r_glob], p1)
            return carry

        lax.fori_loop(0, G_PER_CHUNK, row_group, 0)

    def group(g, carry):
        base = g * L
        acc = b_vec
        for d in range(D):
            acc = acc + prod_t[pl.ds(d * B_PER_W + base, L)]
        out_v[pl.ds(base, L)] = 1.0 / (1.0 + jnp.exp(-acc))
        return carry

    lax.fori_loop(0, N_GROUPS, group, 0)

    pltpu.sync_copy(out_v, out.at[pl.ds(wid * B_PER_W, B_PER_W)])


_gmf = pl.kernel(
    _gmf_body,
    out_type=jax.ShapeDtypeStruct((BATCH,), jnp.float32),
    mesh=plsc.VectorSubcoreMesh(core_axis_name="c", subcore_axis_name="s"),
    compiler_params=pltpu.CompilerParams(
        needs_layout_passes=False, use_tc_tiling_on_sc=True),
    scratch_types=[
        pltpu.VMEM((N_CHUNKS, CHUNK), jnp.int32),
        pltpu.VMEM((N_CHUNKS, CHUNK), jnp.int32),
        pltpu.VMEM((N_CHUNKS, CHUNK), jnp.int32),
        pltpu.VMEM((N_CHUNKS, CHUNK), jnp.int32),
        pltpu.VMEM((B_PER_W,), jnp.int32),
        pltpu.VMEM((B_PER_W,), jnp.int32),
        pltpu.VMEM((CHUNK, 128), jnp.float32),
        pltpu.VMEM((CHUNK, 128), jnp.float32),
        pltpu.VMEM((8, 128), jnp.float32),
        pltpu.VMEM((B_PER_W * D,), jnp.float32),
        pltpu.VMEM((B_PER_W,), jnp.float32),
        pltpu.SemaphoreType.DMA,
    ],
)


@jax.jit
def kernel(user_input, item_input, user_table, item_table, linear_w, linear_b):
    u_idx = user_input.astype(jnp.int32).reshape(NW * N_CHUNKS, CHUNK)
    i_idx = item_input.astype(jnp.int32).reshape(NW * N_CHUNKS, CHUNK)
    u_tab = user_table.reshape(-1, 128)
    i_tab = item_table.reshape(-1, 128)
    w = linear_w.reshape(D)
    wb = jnp.zeros((8, 128), jnp.float32)
    wb = wb.at[0, :L].set(w[:L]).at[1, :L].set(w[L:])
    wb = wb.at[2, :L].set(jnp.broadcast_to(linear_b, (L,)))
    out = _gmf(u_tab, i_tab, u_idx, i_idx, wb)
    return out.reshape(BATCH, 1)


# speedup vs baseline: 3.4668x; 3.4668x over previous
import jax
import jax.numpy as jnp
from jax import lax
from jax.experimental import pallas as pl
from jax.experimental.pallas import tpu as pltpu
from jax.experimental.pallas import tpu_sc as plsc

NC = 2
NS = 16
L = 16
NW = NC * NS

BATCH = 16384
D = 32
B_PER_W = BATCH // NW
CHUNK = 128
N_CHUNKS = B_PER_W // CHUNK
N_BLOCKS = B_PER_W // L
WAVE = 8


def _gmf_body(u_tab, i_tab, u_idx, i_idx, wb,
              out,
              u_idx_v, i_idx_v, u_buck, i_buck, wb_v, prod_t, out_v, sem):
    wid = lax.axis_index("s") * NC + lax.axis_index("c")
    cbase = wid * N_CHUNKS

    pltpu.sync_copy(u_idx.at[pl.ds(cbase, N_CHUNKS)], u_idx_v)
    pltpu.sync_copy(i_idx.at[pl.ds(cbase, N_CHUNKS)], i_idx_v)
    pltpu.sync_copy(wb, wb_v)

    w_lo = wb_v[0, pl.ds(0, L)]
    w_hi = wb_v[1, pl.ds(0, L)]
    b_vec = wb_v[2, pl.ds(0, L)]
    lane = jnp.arange(L, dtype=jnp.int32)
    sc_lo = lane * B_PER_W
    sc_hi = sc_lo + L * B_PER_W
    rows_lo = lane
    rows_hi = lane + L

    def block(g, carry):
        chunk = g // (CHUNK // L)
        off = (g % (CHUNK // L)) * L
        u_iv = u_idx_v[chunk, pl.ds(off, L)]
        i_iv = i_idx_v[chunk, pl.ds(off, L)]
        bu = (u_iv >> 7) << 7
        cu = u_iv & 127
        bi = (i_iv >> 7) << 7
        ci = i_iv & 127

        for wave in range(L // WAVE):
            copies = []
            for j in range(WAVE):
                jj = wave * WAVE + j
                copies.append(pltpu.async_copy(
                    u_tab.at[:, pl.ds(pl.multiple_of(bu[jj], 128), 128)],
                    u_buck.at[pl.ds(j * D, D)], sem))
                copies.append(pltpu.async_copy(
                    i_tab.at[:, pl.ds(pl.multiple_of(bi[jj], 128), 128)],
                    i_buck.at[pl.ds(j * D, D)], sem))
            for c in copies:
                c.wait()
            for j in range(WAVE):
                jj = wave * WAVE + j
                r = g * L + jj
                ucol = jnp.full((L,), cu[jj], jnp.int32)
                icol = jnp.full((L,), ci[jj], jnp.int32)
                u_lo = plsc.load_gather(u_buck, [j * D + rows_lo, ucol])
                u_hi = plsc.load_gather(u_buck, [j * D + rows_hi, ucol])
                i_lo = plsc.load_gather(i_buck, [j * D + rows_lo, icol])
                i_hi = plsc.load_gather(i_buck, [j * D + rows_hi, icol])
                plsc.store_scatter(prod_t, [sc_lo + r], u_lo * i_lo * w_lo)
                plsc.store_scatter(prod_t, [sc_hi + r], u_hi * i_hi * w_hi)
        return carry

    lax.fori_loop(0, N_BLOCKS, block, 0)

    def group(g, carry):
        base = g * L
        acc = b_vec
        for d in range(D):
            acc = acc + prod_t[pl.ds(d * B_PER_W + base, L)]
        out_v[pl.ds(base, L)] = 1.0 / (1.0 + jnp.exp(-acc))
        return carry

    lax.fori_loop(0, N_BLOCKS, group, 0)

    pltpu.sync_copy(out_v, out.at[pl.ds(wid * B_PER_W, B_PER_W)])


_gmf = pl.kernel(
    _gmf_body,
    out_type=jax.ShapeDtypeStruct((BATCH,), jnp.float32),
    mesh=plsc.VectorSubcoreMesh(core_axis_name="c", subcore_axis_name="s"),
    compiler_params=pltpu.CompilerParams(
        needs_layout_passes=False, use_tc_tiling_on_sc=True),
    scratch_types=[
        pltpu.VMEM((N_CHUNKS, CHUNK), jnp.int32),
        pltpu.VMEM((N_CHUNKS, CHUNK), jnp.int32),
        pltpu.VMEM((WAVE * D, 128), jnp.float32),
        pltpu.VMEM((WAVE * D, 128), jnp.float32),
        pltpu.VMEM((8, 128), jnp.float32),
        pltpu.VMEM((B_PER_W * D,), jnp.float32),
        pltpu.VMEM((B_PER_W,), jnp.float32),
        pltpu.SemaphoreType.DMA,
    ],
)


@jax.jit
def kernel(user_input, item_input, user_table, item_table, linear_w, linear_b):
    u_idx = user_input.astype(jnp.int32).reshape(NW * N_CHUNKS, CHUNK)
    i_idx = item_input.astype(jnp.int32).reshape(NW * N_CHUNKS, CHUNK)
    u_tab = user_table.T
    i_tab = item_table.T
    w = linear_w.reshape(D)
    wb = jnp.zeros((8, 128), jnp.float32)
    wb = wb.at[0, :L].set(w[:L]).at[1, :L].set(w[L:])
    wb = wb.at[2, :L].set(jnp.broadcast_to(linear_b, (L,)))
    out = _gmf(u_tab, i_tab, u_idx, i_idx, wb)
    return out.reshape(BATCH, 1)


# speedup vs baseline: 4.0629x; 1.1720x over previous
import jax
import jax.numpy as jnp
from jax import lax
from jax.experimental import pallas as pl
from jax.experimental.pallas import tpu as pltpu
from jax.experimental.pallas import tpu_sc as plsc

NC = 2
NS = 16
L = 16
NW = NC * NS

BATCH = 16384
D = 32
B_PER_W = BATCH // NW
CHUNK = 128
N_CHUNKS = B_PER_W // CHUNK
N_BLOCKS = B_PER_W // L
WAVE = 4
N_WAVES = L // WAVE


def _gmf_body(u_tab, i_tab, u_idx, i_idx, wb,
              out,
              u_idx_v, i_idx_v, u_buck, i_buck, wb_v, prod_t, out_v, sem):
    wid = lax.axis_index("s") * NC + lax.axis_index("c")
    cbase = wid * N_CHUNKS

    pltpu.sync_copy(u_idx.at[pl.ds(cbase, N_CHUNKS)], u_idx_v)
    pltpu.sync_copy(i_idx.at[pl.ds(cbase, N_CHUNKS)], i_idx_v)
    pltpu.sync_copy(wb, wb_v)

    w_lo = wb_v[0, pl.ds(0, L)]
    w_hi = wb_v[1, pl.ds(0, L)]
    b_vec = wb_v[2, pl.ds(0, L)]
    lane = jnp.arange(L, dtype=jnp.int32)
    sc_lo = lane * B_PER_W
    sc_hi = sc_lo + L * B_PER_W
    tr_lo = lane >> 3
    sl_lo = lane & 7
    tr_hi = tr_lo + 2

    def fire(bu, bi, wave):
        slot = (wave % 2) * WAVE
        copies = []
        for j in range(WAVE):
            jj = wave * WAVE + j
            copies.append(pltpu.async_copy(
                u_tab.at[:, :, pl.ds(pl.multiple_of(bu[jj], 128), 128)],
                u_buck.at[pl.ds((slot + j) * 4, 4)], sem))
            copies.append(pltpu.async_copy(
                i_tab.at[:, :, pl.ds(pl.multiple_of(bi[jj], 128), 128)],
                i_buck.at[pl.ds((slot + j) * 4, 4)], sem))
        return copies

    def process(g, cu, ci, wave, copies):
        slot = (wave % 2) * WAVE
        for c in copies:
            c.wait()
        for j in range(WAVE):
            jj = wave * WAVE + j
            r = g * L + jj
            ucol = jnp.full((L,), cu[jj], jnp.int32)
            icol = jnp.full((L,), ci[jj], jnp.int32)
            base = (slot + j) * 4
            u_lo = plsc.load_gather(u_buck, [base + tr_lo, sl_lo, ucol])
            u_hi = plsc.load_gather(u_buck, [base + tr_hi, sl_lo, ucol])
            i_lo = plsc.load_gather(i_buck, [base + tr_lo, sl_lo, icol])
            i_hi = plsc.load_gather(i_buck, [base + tr_hi, sl_lo, icol])
            plsc.store_scatter(prod_t, [sc_lo + r], u_lo * i_lo * w_lo)
            plsc.store_scatter(prod_t, [sc_hi + r], u_hi * i_hi * w_hi)

    def block(g, carry):
        chunk = g // (CHUNK // L)
        off = (g % (CHUNK // L)) * L
        u_iv = u_idx_v[chunk, pl.ds(off, L)]
        i_iv = i_idx_v[chunk, pl.ds(off, L)]
        bu = (u_iv >> 7) << 7
        cu = u_iv & 127
        bi = (i_iv >> 7) << 7
        ci = i_iv & 127

        pending = fire(bu, bi, 0)
        for wave in range(N_WAVES):
            nxt = fire(bu, bi, wave + 1) if wave + 1 < N_WAVES else []
            process(g, cu, ci, wave, pending)
            pending = nxt
        return carry

    lax.fori_loop(0, N_BLOCKS, block, 0)

    def group(g, carry):
        base = g * L
        acc = b_vec
        for d in range(D):
            acc = acc + prod_t[pl.ds(d * B_PER_W + base, L)]
        out_v[pl.ds(base, L)] = 1.0 / (1.0 + jnp.exp(-acc))
        return carry

    lax.fori_loop(0, N_BLOCKS, group, 0)

    pltpu.sync_copy(out_v, out.at[pl.ds(wid * B_PER_W, B_PER_W)])


_gmf = pl.kernel(
    _gmf_body,
    out_type=jax.ShapeDtypeStruct((BATCH,), jnp.float32),
    mesh=plsc.VectorSubcoreMesh(core_axis_name="c", subcore_axis_name="s"),
    compiler_params=pltpu.CompilerParams(
        needs_layout_passes=False, use_tc_tiling_on_sc=True),
    scratch_types=[
        pltpu.VMEM((N_CHUNKS, CHUNK), jnp.int32),
        pltpu.VMEM((N_CHUNKS, CHUNK), jnp.int32),
        pltpu.VMEM((2 * WAVE * 4, 8, 128), jnp.float32),
        pltpu.VMEM((2 * WAVE * 4, 8, 128), jnp.float32),
        pltpu.VMEM((8, 128), jnp.float32),
        pltpu.VMEM((B_PER_W * D,), jnp.float32),
        pltpu.VMEM((B_PER_W,), jnp.float32),
        pltpu.SemaphoreType.DMA,
    ],
)


@jax.jit
def kernel(user_input, item_input, user_table, item_table, linear_w, linear_b):
    u_idx = user_input.astype(jnp.int32).reshape(NW * N_CHUNKS, CHUNK)
    i_idx = item_input.astype(jnp.int32).reshape(NW * N_CHUNKS, CHUNK)
    u_tab = user_table.T.reshape(4, 8, -1)
    i_tab = item_table.T.reshape(4, 8, -1)
    w = linear_w.reshape(D)
    wb = jnp.zeros((8, 128), jnp.float32)
    wb = wb.at[0, :L].set(w[:L]).at[1, :L].set(w[L:])
    wb = wb.at[2, :L].set(jnp.broadcast_to(linear_b, (L,)))
    out = _gmf(u_tab, i_tab, u_idx, i_idx, wb)
    return out.reshape(BATCH, 1)
